# Initial kernel scaffold; baseline (speedup 1.0000x reference)
import jax
import jax.numpy as jnp
from jax.experimental import pallas as pl


def kernel(X, H_prev, edge_index, feat, W_z, b_z, W_r, b_r, W_h, b_h):
    raise NotImplementedError("write your pallas kernel here")


# speedup vs baseline: 3.0403x; 3.0403x over previous
import functools

import jax
import jax.numpy as jnp
from jax import lax
from jax.experimental import pallas as pl
from jax.experimental.pallas import tpu as pltpu
from jax.experimental.pallas import tpu_sc as plsc

_N = 10000
_D = 128
_NTILES = 32
_STRIPE = 640
_ACC_ROWS = _NTILES // 2 * _STRIPE
_CHUNK = 128
_LANES = 16


def _seg_accum(table, idx, dst, nchunks):
  mesh = plsc.VectorSubcoreMesh(core_axis_name="c", subcore_axis_name="s")

  def body(table_hbm, idx_hbm, dst_hbm, out_hbm, idx_v, dst_v, rows_v, acc, sem):
    cid = lax.axis_index("c")
    sid = lax.axis_index("s")
    wid = cid * 16 + sid

    def zrow(i, c):
      r = i // (_D // _LANES)
      k = i % (_D // _LANES)
      rows_v[r, pl.ds(k * _LANES, _LANES)] = jnp.zeros((_LANES,), jnp.float32)
      return c
    lax.fori_loop(0, _CHUNK * (_D // _LANES), zrow, 0)
    for j in range(_STRIPE // _CHUNK):
      pltpu.sync_copy(rows_v, acc.at[pl.ds(sid * _STRIPE + j * _CHUNK, _CHUNK)])

    pltpu.sync_copy(idx_hbm.at[wid], idx_v)
    pltpu.sync_copy(dst_hbm.at[wid], dst_v)
    plsc.subcore_barrier()

    def step(j, c):
      pltpu.async_copy(table_hbm.at[idx_v.at[j]], rows_v, sem).wait()
      pltpu.sync_copy(rows_v, acc.at[dst_v.at[j]], add=True)
      return c
    lax.fori_loop(0, nchunks, step, 0)

    plsc.subcore_barrier()
    pltpu.sync_copy(acc.at[pl.ds(sid * _STRIPE, _STRIPE)],
                    out_hbm.at[cid, pl.ds(sid * _STRIPE, _STRIPE)])

  k = pl.kernel(
      body,
      out_type=jax.ShapeDtypeStruct((2, _ACC_ROWS, _D), jnp.float32),
      mesh=mesh,
      scratch_types=[
          pltpu.VMEM((nchunks, _CHUNK), jnp.int32),
          pltpu.VMEM((nchunks, _CHUNK), jnp.int32),
          pltpu.VMEM((_CHUNK, _D), jnp.float32),
          pltpu.VMEM_SHARED((_ACC_ROWS, _D), jnp.float32),
          pltpu.SemaphoreType.DMA,
      ],
  )
  return k(table, idx, dst)


_BLK = 400

_mm = functools.partial(jnp.dot, preferred_element_type=jnp.float32,
                        precision=lax.Precision.HIGHEST)


def _gates_body(p1, p2, p3, h, wz, wr, bz, br, z_o, m_o, f_o, t1a_o):
  t1a = p1[0] + p1[1]
  t1b = p2[0] + p2[1]
  f = p3[0] + p3[1]
  hh = h[...]
  z = jax.nn.sigmoid(_mm(t1a, wz[:_D]) + _mm(t1b, wz[_D:]) + f + bz[...])
  r = jax.nn.sigmoid(_mm(t1a, wr[:_D]) + _mm(t1b, wr[_D:]) + f + br[...])
  z_o[...] = z
  m_o[...] = r * hh
  f_o[...] = f
  t1a_o[...] = t1a


def _gates(P1, P2, P3, H_prev, W_z, W_r, b_z, b_r):
  part = pl.BlockSpec((2, _BLK, _D), lambda i: (0, i, 0))
  node = pl.BlockSpec((_BLK, _D), lambda i: (i, 0))
  wspec = pl.BlockSpec((2 * _D, _D), lambda i: (0, 0))
  bspec = pl.BlockSpec((1, _D), lambda i: (0, 0))
  out = jax.ShapeDtypeStruct((_N, _D), jnp.float32)
  return pl.pallas_call(
      _gates_body,
      grid=(_N // _BLK,),
      in_specs=[part, part, part, node, wspec, wspec, bspec, bspec],
      out_specs=[node, node, node, node],
      out_shape=[out, out, out, out],
  )(P1, P2, P3, H_prev, W_z, W_r, b_z, b_r)


def _update_body(t1a, p4, f, z, h, wh, bh, h_o):
  t2 = p4[0] + p4[1]
  g = _mm(t1a[...], wh[:_D]) + _mm(t2, wh[_D:]) + f[...] + bh[...]
  h_tilde = jnp.tanh(g)
  zz = z[...]
  h_o[...] = zz * h[...] + (1.0 - zz) * h_tilde


def _update(T1a, P4, F, Z, H_prev, W_h, b_h):
  part = pl.BlockSpec((2, _BLK, _D), lambda i: (0, i, 0))
  node = pl.BlockSpec((_BLK, _D), lambda i: (i, 0))
  wspec = pl.BlockSpec((2 * _D, _D), lambda i: (0, 0))
  bspec = pl.BlockSpec((1, _D), lambda i: (0, 0))
  return pl.pallas_call(
      _update_body,
      grid=(_N // _BLK,),
      in_specs=[node, part, node, node, node, wspec, bspec],
      out_specs=node,
      out_shape=jax.ShapeDtypeStruct((_N, _D), jnp.float32),
  )(T1a, P4, F, Z, H_prev, W_h, b_h)


def kernel(X, H_prev, edge_index, feat, W_z, b_z, W_r, b_r, W_h, b_h):
  e = edge_index.shape[1]
  ept = -(-e // (_NTILES * _CHUNK)) * _CHUNK
  pad = _NTILES * ept - e
  nchunks = ept // _CHUNK

  src = edge_index[0]
  dst = edge_index[1]
  shape3 = (_NTILES, nchunks, _CHUNK)
  src_p = jnp.concatenate(
      [src, jnp.zeros((pad,), jnp.int32)]).reshape(shape3)
  dst_p = jnp.concatenate(
      [dst, jnp.full((pad,), _N, jnp.int32)]).reshape(shape3)
  eid_p = jnp.concatenate(
      [jnp.arange(e, dtype=jnp.int32), jnp.zeros((pad,), jnp.int32)]
  ).reshape(shape3)

  P1 = _seg_accum(X, src_p, dst_p, nchunks)
  P2 = _seg_accum(H_prev, src_p, dst_p, nchunks)
  P3 = _seg_accum(feat, eid_p, dst_p, nchunks)

  Z, M, F, T1a = _gates(P1, P2, P3, H_prev,
                        W_z, W_r, b_z.reshape(1, _D), b_r.reshape(1, _D))

  P4 = _seg_accum(M, src_p, dst_p, nchunks)

  return _update(T1a, P4, F, Z, H_prev, W_h, b_h.reshape(1, _D))


# speedup vs baseline: 3.6988x; 1.2166x over previous
import functools

import jax
import jax.numpy as jnp
from jax import lax
from jax.experimental import pallas as pl
from jax.experimental.pallas import tpu as pltpu
from jax.experimental.pallas import tpu_sc as plsc

_N = 10000
_D = 128
_NTILES = 32
_STRIPE = 640
_ACC_ROWS = _NTILES // 2 * _STRIPE
_CHUNK = 128
_BLKC = 48
_LANES = 16


def _seg_accum(table, idx, dst, nc0, nc1):
  mesh = plsc.VectorSubcoreMesh(core_axis_name="c", subcore_axis_name="s")

  def body(table_hbm, idx_hbm, dst_hbm, out_hbm, idx_v, dst_v, rows0, rows1,
           acc, sem0, sem1):
    cid = lax.axis_index("c")
    sid = lax.axis_index("s")
    wid = cid * 16 + sid

    def zrow(i, c):
      r = i // (_D // _LANES)
      k = i % (_D // _LANES)
      rows0[r, pl.ds(k * _LANES, _LANES)] = jnp.zeros((_LANES,), jnp.float32)
      return c
    lax.fori_loop(0, _CHUNK * (_D // _LANES), zrow, 0)
    for j in range(_STRIPE // _CHUNK):
      pltpu.sync_copy(rows0, acc.at[pl.ds(sid * _STRIPE + j * _CHUNK, _CHUNK)])

    plsc.subcore_barrier()

    nc_mine = jnp.where(cid == 0, nc0, nc1)
    nblocks = (nc_mine + _BLKC - 1) // _BLKC

    def block(b, c):
      base = b * _BLKC
      cnt = jnp.minimum(_BLKC, nc_mine - base)
      pltpu.sync_copy(idx_hbm.at[wid, pl.ds(base, _BLKC)], idx_v)
      pltpu.sync_copy(dst_hbm.at[wid, pl.ds(base, _BLKC)], dst_v)
      pltpu.async_copy(table_hbm.at[idx_v.at[0]], rows0, sem0)

      def step2(i, c2):
        j = i * 2
        pltpu.make_async_copy(table_hbm.at[idx_v.at[j]], rows0, sem0).wait()
        pltpu.async_copy(table_hbm.at[idx_v.at[j + 1]], rows1, sem1)
        pltpu.sync_copy(rows0, acc.at[dst_v.at[j]], add=True)

        pltpu.make_async_copy(table_hbm.at[idx_v.at[j + 1]], rows1, sem1).wait()

        @pl.when(i + 1 < cnt // 2)
        def _():
          pltpu.async_copy(table_hbm.at[idx_v.at[j + 2]], rows0, sem0)

        pltpu.sync_copy(rows1, acc.at[dst_v.at[j + 1]], add=True)
        return c2

      lax.fori_loop(0, cnt // 2, step2, 0)
      return c

    lax.fori_loop(0, nblocks, block, 0)

    plsc.subcore_barrier()
    pltpu.sync_copy(acc.at[pl.ds(sid * _STRIPE, _STRIPE)],
                    out_hbm.at[cid, pl.ds(sid * _STRIPE, _STRIPE)])

  k = pl.kernel(
      body,
      out_type=jax.ShapeDtypeStruct((2, _ACC_ROWS, _D), jnp.float32),
      mesh=mesh,
      scratch_types=[
          pltpu.VMEM((_BLKC, _CHUNK), jnp.int32),
          pltpu.VMEM((_BLKC, _CHUNK), jnp.int32),
          pltpu.VMEM((_CHUNK, _D), jnp.float32),
          pltpu.VMEM((_CHUNK, _D), jnp.float32),
          pltpu.VMEM_SHARED((_ACC_ROWS, _D), jnp.float32),
          pltpu.SemaphoreType.DMA,
          pltpu.SemaphoreType.DMA,
      ],
  )
  return k(table, idx, dst)


_BLK = 400

_mm = functools.partial(jnp.dot, preferred_element_type=jnp.float32,
                        precision=lax.Precision.HIGHEST)


def _gates_body(p1, p2, p3, h, wz, wr, bz, br, z_o, m_o, f_o, t1a_o):
  t1a = p1[0] + p1[1]
  t1b = p2[0] + p2[1]
  f = p3[0] + p3[1]
  hh = h[...]
  z = jax.nn.sigmoid(_mm(t1a, wz[:_D]) + _mm(t1b, wz[_D:]) + f + bz[...])
  r = jax.nn.sigmoid(_mm(t1a, wr[:_D]) + _mm(t1b, wr[_D:]) + f + br[...])
  z_o[...] = z
  m_o[...] = r * hh
  f_o[...] = f
  t1a_o[...] = t1a


def _gates(P1, P2, P3, H_prev, W_z, W_r, b_z, b_r):
  part = pl.BlockSpec((2, _BLK, _D), lambda i: (0, i, 0))
  node = pl.BlockSpec((_BLK, _D), lambda i: (i, 0))
  wspec = pl.BlockSpec((2 * _D, _D), lambda i: (0, 0))
  bspec = pl.BlockSpec((1, _D), lambda i: (0, 0))
  out = jax.ShapeDtypeStruct((_N, _D), jnp.float32)
  return pl.pallas_call(
      _gates_body,
      grid=(_N // _BLK,),
      in_specs=[part, part, part, node, wspec, wspec, bspec, bspec],
      out_specs=[node, node, node, node],
      out_shape=[out, out, out, out],
  )(P1, P2, P3, H_prev, W_z, W_r, b_z, b_r)


def _update_body(t1a, p4, f, z, h, wh, bh, h_o):
  t2 = p4[0] + p4[1]
  g = _mm(t1a[...], wh[:_D]) + _mm(t2, wh[_D:]) + f[...] + bh[...]
  h_tilde = jnp.tanh(g)
  zz = z[...]
  h_o[...] = zz * h[...] + (1.0 - zz) * h_tilde


def _update(T1a, P4, F, Z, H_prev, W_h, b_h):
  part = pl.BlockSpec((2, _BLK, _D), lambda i: (0, i, 0))
  node = pl.BlockSpec((_BLK, _D), lambda i: (i, 0))
  wspec = pl.BlockSpec((2 * _D, _D), lambda i: (0, 0))
  bspec = pl.BlockSpec((1, _D), lambda i: (0, 0))
  return pl.pallas_call(
      _update_body,
      grid=(_N // _BLK,),
      in_specs=[node, part, node, node, node, wspec, bspec],
      out_specs=node,
      out_shape=jax.ShapeDtypeStruct((_N, _D), jnp.float32),
  )(T1a, P4, F, Z, H_prev, W_h, b_h)


def _split_chunks(e):
  total = -(-e // (16 * _CHUNK))
  nc0 = -(-total * 63 // 100)
  nc0 += nc0 % 2
  nc1 = max(total - nc0, 2)
  nc1 += nc1 % 2
  return nc0, nc1


def _pack(arr, pad_val, nc0, nc1):
  e = arr.shape[0]
  e0 = 16 * nc0 * _CHUNK
  e1 = 16 * nc1 * _CHUNK
  ncp = -(-max(nc0, nc1) // _BLKC) * _BLKC
  a = jnp.concatenate([arr, jnp.full((e0 + e1 - e,), pad_val, arr.dtype)])
  a0 = jnp.pad(a[:e0].reshape(16, nc0, _CHUNK),
               ((0, 0), (0, ncp - nc0), (0, 0)))
  a1 = jnp.pad(a[e0:].reshape(16, nc1, _CHUNK),
               ((0, 0), (0, ncp - nc1), (0, 0)))
  return jnp.concatenate([a0, a1], axis=0)


def kernel(X, H_prev, edge_index, feat, W_z, b_z, W_r, b_r, W_h, b_h):
  e = edge_index.shape[1]
  nc0, nc1 = _split_chunks(e)

  src_p = _pack(edge_index[0], 0, nc0, nc1)
  dst_p = _pack(edge_index[1], _N, nc0, nc1)
  eid_p = _pack(jnp.arange(e, dtype=jnp.int32), 0, nc0, nc1)

  P1 = _seg_accum(X, src_p, dst_p, nc0, nc1)
  P2 = _seg_accum(H_prev, src_p, dst_p, nc0, nc1)
  P3 = _seg_accum(feat, eid_p, dst_p, nc0, nc1)

  Z, M, F, T1a = _gates(P1, P2, P3, H_prev,
                        W_z, W_r, b_z.reshape(1, _D), b_r.reshape(1, _D))

  P4 = _seg_accum(M, src_p, dst_p, nc0, nc1)

  return _update(T1a, P4, F, Z, H_prev, W_h, b_h.reshape(1, _D))


# speedup vs baseline: 3.9437x; 1.0662x over previous
import functools

import jax
import jax.numpy as jnp
from jax import lax
from jax.experimental import pallas as pl
from jax.experimental.pallas import tpu as pltpu
from jax.experimental.pallas import tpu_sc as plsc

_N = 10000
_D = 128
_NTILES = 32
_STRIPE = 640
_ACC_ROWS = _NTILES // 2 * _STRIPE
_CHUNK = 128
_BLKC = 48
_LANES = 16


def _seg_accum(table, idx, dst, nc0, nc1):
  mesh = plsc.VectorSubcoreMesh(core_axis_name="c", subcore_axis_name="s")

  def body(table_hbm, idx_hbm, dst_hbm, out_hbm, idx_v, dst_v, rows0, rows1,
           acc, sem0, sem1):
    cid = lax.axis_index("c")
    sid = lax.axis_index("s")
    wid = cid * 16 + sid

    def zrow(i, c):
      r = i // (_D // _LANES)
      k = i % (_D // _LANES)
      rows0[r, pl.ds(k * _LANES, _LANES)] = jnp.zeros((_LANES,), jnp.float32)
      return c
    lax.fori_loop(0, _CHUNK * (_D // _LANES), zrow, 0)
    for j in range(_STRIPE // _CHUNK):
      pltpu.sync_copy(rows0, acc.at[pl.ds(sid * _STRIPE + j * _CHUNK, _CHUNK)])

    plsc.subcore_barrier()

    nc_mine = jnp.where(cid == 0, nc0, nc1)
    nblocks = (nc_mine + _BLKC - 1) // _BLKC

    def block(b, c):
      base = b * _BLKC
      cnt = jnp.minimum(_BLKC, nc_mine - base)
      pltpu.sync_copy(idx_hbm.at[wid, pl.ds(base, _BLKC)], idx_v)
      pltpu.sync_copy(dst_hbm.at[wid, pl.ds(base, _BLKC)], dst_v)
      pltpu.async_copy(table_hbm.at[idx_v.at[0]], rows0, sem0)

      def step2(i, c2):
        j = i * 2
        pltpu.make_async_copy(table_hbm.at[idx_v.at[j]], rows0, sem0).wait()
        pltpu.async_copy(table_hbm.at[idx_v.at[j + 1]], rows1, sem1)
        pltpu.sync_copy(rows0, acc.at[dst_v.at[j]], add=True)

        pltpu.make_async_copy(table_hbm.at[idx_v.at[j + 1]], rows1, sem1).wait()

        @pl.when(i + 1 < cnt // 2)
        def _():
          pltpu.async_copy(table_hbm.at[idx_v.at[j + 2]], rows0, sem0)

        pltpu.sync_copy(rows1, acc.at[dst_v.at[j + 1]], add=True)
        return c2

      lax.fori_loop(0, cnt // 2, step2, 0)
      return c

    lax.fori_loop(0, nblocks, block, 0)

    plsc.subcore_barrier()
    pltpu.sync_copy(acc.at[pl.ds(sid * _STRIPE, _STRIPE)],
                    out_hbm.at[cid, pl.ds(sid * _STRIPE, _STRIPE)])

  k = pl.kernel(
      body,
      out_type=jax.ShapeDtypeStruct((2, _ACC_ROWS, _D), jnp.float32),
      mesh=mesh,
      scratch_types=[
          pltpu.VMEM((_BLKC, _CHUNK), jnp.int32),
          pltpu.VMEM((_BLKC, _CHUNK), jnp.int32),
          pltpu.VMEM((_CHUNK, _D), jnp.float32),
          pltpu.VMEM((_CHUNK, _D), jnp.float32),
          pltpu.VMEM_SHARED((_ACC_ROWS, _D), jnp.float32),
          pltpu.SemaphoreType.DMA,
          pltpu.SemaphoreType.DMA,
      ],
  )
  return k(table, idx, dst)


_BLK = 400

_mm = functools.partial(jnp.dot, preferred_element_type=jnp.float32,
                        precision=lax.Precision.HIGHEST)


def _gates_body(p1, p2, p3, h, wz, wr, bz, br, z_o, m_o, f_o, t1a_o):
  t1a = p1[0] + p1[1]
  t1b = p2[0] + p2[1]
  f = p3[0] + p3[1]
  hh = h[...]
  z = jax.nn.sigmoid(_mm(t1a, wz[:_D]) + _mm(t1b, wz[_D:]) + f + bz[...])
  r = jax.nn.sigmoid(_mm(t1a, wr[:_D]) + _mm(t1b, wr[_D:]) + f + br[...])
  z_o[...] = z
  m_o[...] = r * hh
  f_o[...] = f
  t1a_o[...] = t1a


def _gates(P1, P2, P3, H_prev, W_z, W_r, b_z, b_r):
  part = pl.BlockSpec((2, _BLK, _D), lambda i: (0, i, 0))
  node = pl.BlockSpec((_BLK, _D), lambda i: (i, 0))
  wspec = pl.BlockSpec((2 * _D, _D), lambda i: (0, 0))
  bspec = pl.BlockSpec((1, _D), lambda i: (0, 0))
  out = jax.ShapeDtypeStruct((_N, _D), jnp.float32)
  return pl.pallas_call(
      _gates_body,
      grid=(_N // _BLK,),
      in_specs=[part, part, part, node, wspec, wspec, bspec, bspec],
      out_specs=[node, node, node, node],
      out_shape=[out, out, out, out],
  )(P1, P2, P3, H_prev, W_z, W_r, b_z, b_r)


def _update_body(t1a, p4, f, z, h, wh, bh, h_o):
  t2 = p4[0] + p4[1]
  g = _mm(t1a[...], wh[:_D]) + _mm(t2, wh[_D:]) + f[...] + bh[...]
  h_tilde = jnp.tanh(g)
  zz = z[...]
  h_o[...] = zz * h[...] + (1.0 - zz) * h_tilde


def _update(T1a, P4, F, Z, H_prev, W_h, b_h):
  part = pl.BlockSpec((2, _BLK, _D), lambda i: (0, i, 0))
  node = pl.BlockSpec((_BLK, _D), lambda i: (i, 0))
  wspec = pl.BlockSpec((2 * _D, _D), lambda i: (0, 0))
  bspec = pl.BlockSpec((1, _D), lambda i: (0, 0))
  return pl.pallas_call(
      _update_body,
      grid=(_N // _BLK,),
      in_specs=[node, part, node, node, node, wspec, bspec],
      out_specs=node,
      out_shape=jax.ShapeDtypeStruct((_N, _D), jnp.float32),
  )(T1a, P4, F, Z, H_prev, W_h, b_h)


def _split_chunks(e):
  total = -(-e // (16 * _CHUNK))
  nc0 = -(-total * 72 // 100)
  nc0 += nc0 % 2
  nc1 = max(total - nc0, 2)
  nc1 += nc1 % 2
  return nc0, nc1


def _pack(arr, pad_val, nc0, nc1):
  e = arr.shape[0]
  e0 = 16 * nc0 * _CHUNK
  e1 = 16 * nc1 * _CHUNK
  ncp = -(-max(nc0, nc1) // _BLKC) * _BLKC
  a = jnp.concatenate([arr, jnp.full((e0 + e1 - e,), pad_val, arr.dtype)])
  a0 = jnp.pad(a[:e0].reshape(16, nc0, _CHUNK),
               ((0, 0), (0, ncp - nc0), (0, 0)))
  a1 = jnp.pad(a[e0:].reshape(16, nc1, _CHUNK),
               ((0, 0), (0, ncp - nc1), (0, 0)))
  return jnp.concatenate([a0, a1], axis=0)


def kernel(X, H_prev, edge_index, feat, W_z, b_z, W_r, b_r, W_h, b_h):
  e = edge_index.shape[1]
  nc0, nc1 = _split_chunks(e)

  src_p = _pack(edge_index[0], 0, nc0, nc1)
  dst_p = _pack(edge_index[1], _N, nc0, nc1)
  eid_p = _pack(jnp.arange(e, dtype=jnp.int32), 0, nc0, nc1)

  P1 = _seg_accum(X, src_p, dst_p, nc0, nc1)
  P2 = _seg_accum(H_prev, src_p, dst_p, nc0, nc1)
  P3 = _seg_accum(feat, eid_p, dst_p, nc0, nc1)

  Z, M, F, T1a = _gates(P1, P2, P3, H_prev,
                        W_z, W_r, b_z.reshape(1, _D), b_r.reshape(1, _D))

  P4 = _seg_accum(M, src_p, dst_p, nc0, nc1)

  return _update(T1a, P4, F, Z, H_prev, W_h, b_h.reshape(1, _D))


# speedup vs baseline: 3.9670x; 1.0059x over previous
import functools

import jax
import jax.numpy as jnp
from jax import lax
from jax.experimental import pallas as pl
from jax.experimental.pallas import tpu as pltpu
from jax.experimental.pallas import tpu_sc as plsc

_N = 10000
_D = 128
_NTILES = 32
_STRIPE = 640
_ACC_ROWS = _NTILES // 2 * _STRIPE
_CHUNK = 128
_BLKC = 48
_LANES = 16


def _seg_accum(table, idx, dst, nc0, nc1):
  mesh = plsc.VectorSubcoreMesh(core_axis_name="c", subcore_axis_name="s")

  def body(table_hbm, idx_hbm, dst_hbm, out_hbm, idx_v, dst_v, rows0, rows1,
           acc, sem0, sem1):
    cid = lax.axis_index("c")
    sid = lax.axis_index("s")
    wid = cid * 16 + sid

    with jax.named_scope("sc_zero"):
      def zrow(i, c):
        r = i // (_D // _LANES)
        k = i % (_D // _LANES)
        rows0[r, pl.ds(k * _LANES, _LANES)] = jnp.zeros((_LANES,), jnp.float32)
        return c
      lax.fori_loop(0, _CHUNK * (_D // _LANES), zrow, 0)
      for j in range(_STRIPE // _CHUNK):
        pltpu.sync_copy(rows0, acc.at[pl.ds(sid * _STRIPE + j * _CHUNK, _CHUNK)])

      plsc.subcore_barrier()

    nc_mine = jnp.where(cid == 0, nc0, nc1)
    nblocks = (nc_mine + _BLKC - 1) // _BLKC
    sc_edges = jax.named_scope("sc_edges")
    sc_edges.__enter__()

    def block(b, c):
      base = b * _BLKC
      cnt = jnp.minimum(_BLKC, nc_mine - base)
      pltpu.sync_copy(idx_hbm.at[wid, pl.ds(base, _BLKC)], idx_v)
      pltpu.sync_copy(dst_hbm.at[wid, pl.ds(base, _BLKC)], dst_v)
      pltpu.async_copy(table_hbm.at[idx_v.at[0]], rows0, sem0)

      def step2(i, c2):
        j = i * 2
        pltpu.make_async_copy(table_hbm.at[idx_v.at[j]], rows0, sem0).wait()
        pltpu.async_copy(table_hbm.at[idx_v.at[j + 1]], rows1, sem1)
        pltpu.sync_copy(rows0, acc.at[dst_v.at[j]], add=True)

        pltpu.make_async_copy(table_hbm.at[idx_v.at[j + 1]], rows1, sem1).wait()

        @pl.when(i + 1 < cnt // 2)
        def _():
          pltpu.async_copy(table_hbm.at[idx_v.at[j + 2]], rows0, sem0)

        pltpu.sync_copy(rows1, acc.at[dst_v.at[j + 1]], add=True)
        return c2

      lax.fori_loop(0, cnt // 2, step2, 0)
      return c

    lax.fori_loop(0, nblocks, block, 0)
    sc_edges.__exit__(None, None, None)

    with jax.named_scope("sc_flush"):
      plsc.subcore_barrier()
      pltpu.sync_copy(acc.at[pl.ds(sid * _STRIPE, _STRIPE)],
                      out_hbm.at[cid, pl.ds(sid * _STRIPE, _STRIPE)])

  k = pl.kernel(
      body,
      out_type=jax.ShapeDtypeStruct((2, _ACC_ROWS, _D), jnp.float32),
      mesh=mesh,
      scratch_types=[
          pltpu.VMEM((_BLKC, _CHUNK), jnp.int32),
          pltpu.VMEM((_BLKC, _CHUNK), jnp.int32),
          pltpu.VMEM((_CHUNK, _D), jnp.float32),
          pltpu.VMEM((_CHUNK, _D), jnp.float32),
          pltpu.VMEM_SHARED((_ACC_ROWS, _D), jnp.float32),
          pltpu.SemaphoreType.DMA,
          pltpu.SemaphoreType.DMA,
      ],
  )
  return k(table, idx, dst)


_BLK = 400

_mm = functools.partial(jnp.dot, preferred_element_type=jnp.float32,
                        precision=lax.Precision.HIGHEST)


def _gates_body(p1, p2, p3, h, wz, wr, bz, br, z_o, m_o, f_o, t1a_o):
  t1a = p1[0] + p1[1]
  t1b = p2[0] + p2[1]
  f = p3[0] + p3[1]
  hh = h[...]
  z = jax.nn.sigmoid(_mm(t1a, wz[:_D]) + _mm(t1b, wz[_D:]) + f + bz[...])
  r = jax.nn.sigmoid(_mm(t1a, wr[:_D]) + _mm(t1b, wr[_D:]) + f + br[...])
  z_o[...] = z
  m_o[...] = r * hh
  f_o[...] = f
  t1a_o[...] = t1a


def _gates(P1, P2, P3, H_prev, W_z, W_r, b_z, b_r):
  part = pl.BlockSpec((2, _BLK, _D), lambda i: (0, i, 0))
  node = pl.BlockSpec((_BLK, _D), lambda i: (i, 0))
  wspec = pl.BlockSpec((2 * _D, _D), lambda i: (0, 0))
  bspec = pl.BlockSpec((1, _D), lambda i: (0, 0))
  out = jax.ShapeDtypeStruct((_N, _D), jnp.float32)
  return pl.pallas_call(
      _gates_body,
      grid=(_N // _BLK,),
      in_specs=[part, part, part, node, wspec, wspec, bspec, bspec],
      out_specs=[node, node, node, node],
      out_shape=[out, out, out, out],
  )(P1, P2, P3, H_prev, W_z, W_r, b_z, b_r)


def _update_body(t1a, p4, f, z, h, wh, bh, h_o):
  t2 = p4[0] + p4[1]
  g = _mm(t1a[...], wh[:_D]) + _mm(t2, wh[_D:]) + f[...] + bh[...]
  h_tilde = jnp.tanh(g)
  zz = z[...]
  h_o[...] = zz * h[...] + (1.0 - zz) * h_tilde


def _update(T1a, P4, F, Z, H_prev, W_h, b_h):
  part = pl.BlockSpec((2, _BLK, _D), lambda i: (0, i, 0))
  node = pl.BlockSpec((_BLK, _D), lambda i: (i, 0))
  wspec = pl.BlockSpec((2 * _D, _D), lambda i: (0, 0))
  bspec = pl.BlockSpec((1, _D), lambda i: (0, 0))
  return pl.pallas_call(
      _update_body,
      grid=(_N // _BLK,),
      in_specs=[node, part, node, node, node, wspec, bspec],
      out_specs=node,
      out_shape=jax.ShapeDtypeStruct((_N, _D), jnp.float32),
  )(T1a, P4, F, Z, H_prev, W_h, b_h)


def _split_chunks(e):
  total = -(-e // (16 * _CHUNK))
  nc0 = -(-total * 72 // 100)
  nc0 += nc0 % 2
  nc1 = max(total - nc0, 2)
  nc1 += nc1 % 2
  return nc0, nc1


def _pack(arr, pad_val, nc0, nc1):
  e = arr.shape[0]
  e0 = 16 * nc0 * _CHUNK
  e1 = 16 * nc1 * _CHUNK
  ncp = -(-max(nc0, nc1) // _BLKC) * _BLKC
  a = jnp.concatenate([arr, jnp.full((e0 + e1 - e,), pad_val, arr.dtype)])
  a0 = jnp.pad(a[:e0].reshape(16, nc0, _CHUNK),
               ((0, 0), (0, ncp - nc0), (0, 0)))
  a1 = jnp.pad(a[e0:].reshape(16, nc1, _CHUNK),
               ((0, 0), (0, ncp - nc1), (0, 0)))
  return jnp.concatenate([a0, a1], axis=0)


def kernel(X, H_prev, edge_index, feat, W_z, b_z, W_r, b_r, W_h, b_h):
  e = edge_index.shape[1]
  nc0, nc1 = _split_chunks(e)

  src_p = _pack(edge_index[0], 0, nc0, nc1)
  dst_p = _pack(edge_index[1], _N, nc0, nc1)
  eid_p = _pack(jnp.arange(e, dtype=jnp.int32), 0, nc0, nc1)

  P1 = _seg_accum(X, src_p, dst_p, nc0, nc1)
  P2 = _seg_accum(H_prev, src_p, dst_p, nc0, nc1)
  P3 = _seg_accum(feat, eid_p, dst_p, nc0, nc1)

  Z, M, F, T1a = _gates(P1, P2, P3, H_prev,
                        W_z, W_r, b_z.reshape(1, _D), b_r.reshape(1, _D))

  P4 = _seg_accum(M, src_p, dst_p, nc0, nc1)

  return _update(T1a, P4, F, Z, H_prev, W_h, b_h.reshape(1, _D))


# speedup vs baseline: 6.7235x; 1.6948x over previous
import functools

import jax
import jax.numpy as jnp
from jax import lax
from jax.experimental import pallas as pl
from jax.experimental.pallas import tpu as pltpu
from jax.experimental.pallas import tpu_sc as plsc

_N = 10000
_D = 128
_NTILES = 32
_STRIPE = 640
_ACC_ROWS = _NTILES // 2 * _STRIPE
_CHUNK = 128
_BLKC = 48
_LANES = 16


def _seg_accum(table, idx, dst, nc0, nc1):
  mesh = plsc.VectorSubcoreMesh(core_axis_name="c", subcore_axis_name="s")

  def body(table_hbm, idx_hbm, dst_hbm, out_hbm, idx_v, dst_v, rows0, rows1,
           acc, sem0, sem1):
    cid = lax.axis_index("c")
    sid = lax.axis_index("s")
    wid = cid * 16 + sid

    with jax.named_scope("sc_zero"):
      def zrow(i, c):
        r = i // (_D // _LANES)
        k = i % (_D // _LANES)
        rows0[r, pl.ds(k * _LANES, _LANES)] = jnp.zeros((_LANES,), jnp.float32)
        return c
      lax.fori_loop(0, _CHUNK * (_D // _LANES), zrow, 0)
      for j in range(_STRIPE // _CHUNK):
        pltpu.sync_copy(rows0, acc.at[pl.ds(sid * _STRIPE + j * _CHUNK, _CHUNK)])

      plsc.subcore_barrier()

    nc_mine = jnp.where(cid == 0, nc0, nc1)
    nblocks = (nc_mine + _BLKC - 1) // _BLKC
    sc_edges = jax.named_scope("sc_edges")
    sc_edges.__enter__()

    def block(b, c):
      base = b * _BLKC
      cnt = jnp.minimum(_BLKC, nc_mine - base)
      pltpu.sync_copy(idx_hbm.at[wid, pl.ds(base, _BLKC)], idx_v)
      pltpu.sync_copy(dst_hbm.at[wid, pl.ds(base, _BLKC)], dst_v)
      pltpu.async_copy(table_hbm.at[idx_v.at[0]], rows0, sem0)

      def step2(i, c2):
        j = i * 2
        pltpu.make_async_copy(table_hbm.at[idx_v.at[j]], rows0, sem0).wait()
        pltpu.async_copy(table_hbm.at[idx_v.at[j + 1]], rows1, sem1)
        pltpu.sync_copy(rows0, acc.at[dst_v.at[j]], add=True)

        pltpu.make_async_copy(table_hbm.at[idx_v.at[j + 1]], rows1, sem1).wait()

        @pl.when(i + 1 < cnt // 2)
        def _():
          pltpu.async_copy(table_hbm.at[idx_v.at[j + 2]], rows0, sem0)

        pltpu.sync_copy(rows1, acc.at[dst_v.at[j + 1]], add=True)
        return c2

      lax.fori_loop(0, cnt // 2, step2, 0)
      return c

    lax.fori_loop(0, nblocks, block, 0)
    sc_edges.__exit__(None, None, None)

    with jax.named_scope("sc_flush"):
      plsc.subcore_barrier()
      pltpu.sync_copy(acc.at[pl.ds(sid * _STRIPE, _STRIPE)],
                      out_hbm.at[cid, pl.ds(sid * _STRIPE, _STRIPE)])

  k = pl.kernel(
      body,
      out_type=jax.ShapeDtypeStruct((2, _ACC_ROWS, _D), jnp.float32),
      mesh=mesh,
      scratch_types=[
          pltpu.VMEM((_BLKC, _CHUNK), jnp.int32),
          pltpu.VMEM((_BLKC, _CHUNK), jnp.int32),
          pltpu.VMEM((_CHUNK, _D), jnp.float32),
          pltpu.VMEM((_CHUNK, _D), jnp.float32),
          pltpu.VMEM_SHARED((_ACC_ROWS, _D), jnp.float32),
          pltpu.SemaphoreType.DMA,
          pltpu.SemaphoreType.DMA,
      ],
  )
  return k(table, idx, dst)


_BLK = 400

_mm = functools.partial(jnp.dot, preferred_element_type=jnp.float32,
                        precision=lax.Precision.HIGHEST)


def _gates_body(p1, p2, p3, h, wz, wr, bz, br, z_o, m_o, f_o, t1a_o):
  t1a = p1[0] + p1[1]
  t1b = p2[0] + p2[1]
  f = p3[0] + p3[1]
  hh = h[...]
  z = jax.nn.sigmoid(_mm(t1a, wz[:_D]) + _mm(t1b, wz[_D:]) + f + bz[...])
  r = jax.nn.sigmoid(_mm(t1a, wr[:_D]) + _mm(t1b, wr[_D:]) + f + br[...])
  z_o[...] = z
  m_o[...] = r * hh
  f_o[...] = f
  t1a_o[...] = t1a


def _gates(P1, P2, P3, H_prev, W_z, W_r, b_z, b_r):
  part = pl.BlockSpec((2, _BLK, _D), lambda i: (0, i, 0))
  node = pl.BlockSpec((_BLK, _D), lambda i: (i, 0))
  wspec = pl.BlockSpec((2 * _D, _D), lambda i: (0, 0))
  bspec = pl.BlockSpec((1, _D), lambda i: (0, 0))
  out = jax.ShapeDtypeStruct((_N, _D), jnp.float32)
  return pl.pallas_call(
      _gates_body,
      grid=(_N // _BLK,),
      in_specs=[part, part, part, node, wspec, wspec, bspec, bspec],
      out_specs=[node, node, node, node],
      out_shape=[out, out, out, out],
  )(P1, P2, P3, H_prev, W_z, W_r, b_z, b_r)


def _update_body(t1a, p4, f, z, h, wh, bh, h_o):
  t2 = p4[0] + p4[1]
  g = _mm(t1a[...], wh[:_D]) + _mm(t2, wh[_D:]) + f[...] + bh[...]
  h_tilde = jnp.tanh(g)
  zz = z[...]
  h_o[...] = zz * h[...] + (1.0 - zz) * h_tilde


def _update(T1a, P4, F, Z, H_prev, W_h, b_h):
  part = pl.BlockSpec((2, _BLK, _D), lambda i: (0, i, 0))
  node = pl.BlockSpec((_BLK, _D), lambda i: (i, 0))
  wspec = pl.BlockSpec((2 * _D, _D), lambda i: (0, 0))
  bspec = pl.BlockSpec((1, _D), lambda i: (0, 0))
  return pl.pallas_call(
      _update_body,
      grid=(_N // _BLK,),
      in_specs=[node, part, node, node, node, wspec, bspec],
      out_specs=node,
      out_shape=jax.ShapeDtypeStruct((_N, _D), jnp.float32),
  )(T1a, P4, F, Z, H_prev, W_h, b_h)


def _split_chunks(e):
  total = -(-e // (16 * _CHUNK))
  nc0 = -(-total // 2)
  nc0 += nc0 % 2
  nc1 = max(total - nc0, 2)
  nc1 += nc1 % 2
  return nc0, nc1


def _pack(arr, pad_arr, nc0, nc1):
  e0 = 16 * nc0 * _CHUNK
  e1 = 16 * nc1 * _CHUNK
  ncp = -(-max(nc0, nc1) // _BLKC) * _BLKC
  a = jnp.concatenate([arr, pad_arr])
  a0 = jnp.pad(a[:e0].reshape(16, nc0, _CHUNK),
               ((0, 0), (0, ncp - nc0), (0, 0)))
  a1 = jnp.pad(a[e0:].reshape(16, nc1, _CHUNK),
               ((0, 0), (0, ncp - nc1), (0, 0)))
  return jnp.concatenate([a0, a1], axis=0)


def kernel(X, H_prev, edge_index, feat, W_z, b_z, W_r, b_r, W_h, b_h):
  e = edge_index.shape[1]
  nc0, nc1 = _split_chunks(e)

  pad = 16 * (nc0 + nc1) * _CHUNK - e
  ppos = jnp.arange(pad, dtype=jnp.int32)
  src_p = _pack(edge_index[0], ppos % _N, nc0, nc1)
  dst_p = _pack(edge_index[1], _N + ppos % (_ACC_ROWS - _N), nc0, nc1)
  eid_p = _pack(jnp.arange(e, dtype=jnp.int32), ppos % e, nc0, nc1)

  P1 = _seg_accum(X, src_p, dst_p, nc0, nc1)
  P2 = _seg_accum(H_prev, src_p, dst_p, nc0, nc1)
  P3 = _seg_accum(feat, eid_p, dst_p, nc0, nc1)

  Z, M, F, T1a = _gates(P1, P2, P3, H_prev,
                        W_z, W_r, b_z.reshape(1, _D), b_r.reshape(1, _D))

  P4 = _seg_accum(M, src_p, dst_p, nc0, nc1)

  return _update(T1a, P4, F, Z, H_prev, W_h, b_h.reshape(1, _D))


# speedup vs baseline: 6.7802x; 1.0084x over previous
import functools

import jax
import jax.numpy as jnp
from jax import lax
from jax.experimental import pallas as pl
from jax.experimental.pallas import tpu as pltpu
from jax.experimental.pallas import tpu_sc as plsc

_N = 10000
_D = 128
_NTILES = 32
_STRIPE = 640
_ACC_ROWS = _NTILES // 2 * _STRIPE
_CHUNK = 128
_BLKC = 48
_LANES = 16


def _seg_accum(table, idx, dst, nc0, nc1):
  mesh = plsc.VectorSubcoreMesh(core_axis_name="c", subcore_axis_name="s")

  def body(table_hbm, idx_hbm, dst_hbm, out_hbm, idx_v, dst_v, rows0, rows1,
           acc, sem0, sem1):
    cid = lax.axis_index("c")
    sid = lax.axis_index("s")
    wid = cid * 16 + sid

    def zrow(i, c):
      r = i // (_D // _LANES)
      k = i % (_D // _LANES)
      rows0[r, pl.ds(k * _LANES, _LANES)] = jnp.zeros((_LANES,), jnp.float32)
      return c
    lax.fori_loop(0, _CHUNK * (_D // _LANES), zrow, 0)
    for j in range(_STRIPE // _CHUNK):
      pltpu.sync_copy(rows0, acc.at[pl.ds(sid * _STRIPE + j * _CHUNK, _CHUNK)])

    plsc.subcore_barrier()

    nc_mine = jnp.where(cid == 0, nc0, nc1)
    nblocks = (nc_mine + _BLKC - 1) // _BLKC

    def block(b, c):
      base = b * _BLKC
      cnt = jnp.minimum(_BLKC, nc_mine - base)
      pltpu.sync_copy(idx_hbm.at[wid, pl.ds(base, _BLKC)], idx_v)
      pltpu.sync_copy(dst_hbm.at[wid, pl.ds(base, _BLKC)], dst_v)
      pltpu.async_copy(table_hbm.at[idx_v.at[0]], rows0, sem0)

      def step2(i, c2):
        j = i * 2
        pltpu.make_async_copy(table_hbm.at[idx_v.at[j]], rows0, sem0).wait()
        pltpu.async_copy(table_hbm.at[idx_v.at[j + 1]], rows1, sem1)
        pltpu.sync_copy(rows0, acc.at[dst_v.at[j]], add=True)

        pltpu.make_async_copy(table_hbm.at[idx_v.at[j + 1]], rows1, sem1).wait()
        pltpu.async_copy(table_hbm.at[idx_v.at[j + 2]], rows0, sem0)
        pltpu.sync_copy(rows1, acc.at[dst_v.at[j + 1]], add=True)
        return c2

      lax.fori_loop(0, cnt // 2 - 1, step2, 0)

      jl = cnt - 2
      pltpu.make_async_copy(table_hbm.at[idx_v.at[jl]], rows0, sem0).wait()
      pltpu.async_copy(table_hbm.at[idx_v.at[jl + 1]], rows1, sem1)
      pltpu.sync_copy(rows0, acc.at[dst_v.at[jl]], add=True)
      pltpu.make_async_copy(table_hbm.at[idx_v.at[jl + 1]], rows1, sem1).wait()
      pltpu.sync_copy(rows1, acc.at[dst_v.at[jl + 1]], add=True)
      return c

    lax.fori_loop(0, nblocks, block, 0)

    plsc.subcore_barrier()
    pltpu.sync_copy(acc.at[pl.ds(sid * _STRIPE, _STRIPE)],
                    out_hbm.at[cid, pl.ds(sid * _STRIPE, _STRIPE)])

  k = pl.kernel(
      body,
      out_type=jax.ShapeDtypeStruct((2, _ACC_ROWS, _D), jnp.float32),
      mesh=mesh,
      scratch_types=[
          pltpu.VMEM((_BLKC, _CHUNK), jnp.int32),
          pltpu.VMEM((_BLKC, _CHUNK), jnp.int32),
          pltpu.VMEM((_CHUNK, _D), jnp.float32),
          pltpu.VMEM((_CHUNK, _D), jnp.float32),
          pltpu.VMEM_SHARED((_ACC_ROWS, _D), jnp.float32),
          pltpu.SemaphoreType.DMA,
          pltpu.SemaphoreType.DMA,
      ],
  )
  return k(table, idx, dst)


_BLK = 400

_mm = functools.partial(jnp.dot, preferred_element_type=jnp.float32,
                        precision=lax.Precision.HIGHEST)


def _gates_body(p1, p2, p3, h, wz, wr, bz, br, z_o, m_o, f_o, t1a_o):
  t1a = p1[0] + p1[1]
  t1b = p2[0] + p2[1]
  f = p3[0] + p3[1]
  hh = h[...]
  z = jax.nn.sigmoid(_mm(t1a, wz[:_D]) + _mm(t1b, wz[_D:]) + f + bz[...])
  r = jax.nn.sigmoid(_mm(t1a, wr[:_D]) + _mm(t1b, wr[_D:]) + f + br[...])
  z_o[...] = z
  m_o[...] = r * hh
  f_o[...] = f
  t1a_o[...] = t1a


def _gates(P1, P2, P3, H_prev, W_z, W_r, b_z, b_r):
  part = pl.BlockSpec((2, _BLK, _D), lambda i: (0, i, 0))
  node = pl.BlockSpec((_BLK, _D), lambda i: (i, 0))
  wspec = pl.BlockSpec((2 * _D, _D), lambda i: (0, 0))
  bspec = pl.BlockSpec((1, _D), lambda i: (0, 0))
  out = jax.ShapeDtypeStruct((_N, _D), jnp.float32)
  return pl.pallas_call(
      _gates_body,
      grid=(_N // _BLK,),
      in_specs=[part, part, part, node, wspec, wspec, bspec, bspec],
      out_specs=[node, node, node, node],
      out_shape=[out, out, out, out],
  )(P1, P2, P3, H_prev, W_z, W_r, b_z, b_r)


def _update_body(t1a, p4, f, z, h, wh, bh, h_o):
  t2 = p4[0] + p4[1]
  g = _mm(t1a[...], wh[:_D]) + _mm(t2, wh[_D:]) + f[...] + bh[...]
  h_tilde = jnp.tanh(g)
  zz = z[...]
  h_o[...] = zz * h[...] + (1.0 - zz) * h_tilde


def _update(T1a, P4, F, Z, H_prev, W_h, b_h):
  part = pl.BlockSpec((2, _BLK, _D), lambda i: (0, i, 0))
  node = pl.BlockSpec((_BLK, _D), lambda i: (i, 0))
  wspec = pl.BlockSpec((2 * _D, _D), lambda i: (0, 0))
  bspec = pl.BlockSpec((1, _D), lambda i: (0, 0))
  return pl.pallas_call(
      _update_body,
      grid=(_N // _BLK,),
      in_specs=[node, part, node, node, node, wspec, bspec],
      out_specs=node,
      out_shape=jax.ShapeDtypeStruct((_N, _D), jnp.float32),
  )(T1a, P4, F, Z, H_prev, W_h, b_h)


def _split_chunks(e):
  total = -(-e // (16 * _CHUNK))
  nc0 = -(-total // 2)
  nc0 += nc0 % 2
  nc1 = max(total - nc0, 2)
  nc1 += nc1 % 2
  return nc0, nc1


def _pack(arr, pad_arr, nc0, nc1):
  e0 = 16 * nc0 * _CHUNK
  e1 = 16 * nc1 * _CHUNK
  ncp = -(-max(nc0, nc1) // _BLKC) * _BLKC
  a = jnp.concatenate([arr, pad_arr])
  a0 = jnp.pad(a[:e0].reshape(16, nc0, _CHUNK),
               ((0, 0), (0, ncp - nc0), (0, 0)))
  a1 = jnp.pad(a[e0:].reshape(16, nc1, _CHUNK),
               ((0, 0), (0, ncp - nc1), (0, 0)))
  return jnp.concatenate([a0, a1], axis=0)


def kernel(X, H_prev, edge_index, feat, W_z, b_z, W_r, b_r, W_h, b_h):
  e = edge_index.shape[1]
  nc0, nc1 = _split_chunks(e)

  pad = 16 * (nc0 + nc1) * _CHUNK - e
  ppos = jnp.arange(pad, dtype=jnp.int32)
  src_p = _pack(edge_index[0], ppos % _N, nc0, nc1)
  dst_p = _pack(edge_index[1], _N + ppos % (_ACC_ROWS - _N), nc0, nc1)
  eid_p = _pack(jnp.arange(e, dtype=jnp.int32), ppos % e, nc0, nc1)

  P1 = _seg_accum(X, src_p, dst_p, nc0, nc1)
  P2 = _seg_accum(H_prev, src_p, dst_p, nc0, nc1)
  P3 = _seg_accum(feat, eid_p, dst_p, nc0, nc1)

  Z, M, F, T1a = _gates(P1, P2, P3, H_prev,
                        W_z, W_r, b_z.reshape(1, _D), b_r.reshape(1, _D))

  P4 = _seg_accum(M, src_p, dst_p, nc0, nc1)

  return _update(T1a, P4, F, Z, H_prev, W_h, b_h.reshape(1, _D))
